# baseline (speedup 1.0000x reference)
import functools

import jax
import jax.numpy as jnp
from jax import lax
from jax.experimental import pallas as pl
from jax.experimental.pallas import tpu as pltpu
from jax.experimental.pallas import tpu_sc as plsc

N = 100000
M = 20000
K = 16
OBS = 16
NW = 32
MP = 20480
BW = MP // NW
QC = 64
NCH = BW // QC
CW = QC * K
G = QC // 16
ROWS_W = BW * K
IDX_ROWS = ROWS_W // 128
RPC = CW // 128


def _build_sc_kernel():
    mesh = plsc.VectorSubcoreMesh(core_axis_name="c", subcore_axis_name="s")

    @functools.partial(
        pl.kernel,
        out_type=jax.ShapeDtypeStruct((NW, 16, BW), jnp.float32),
        mesh=mesh,
        scratch_types=[
            pltpu.VMEM((IDX_ROWS, 128), jnp.int32),
            pltpu.VMEM((ROWS_W,), jnp.float32),
            pltpu.VMEM((ROWS_W,), jnp.float32),
            pltpu.VMEM((ROWS_W,), jnp.float32),
            pltpu.VMEM((BW,), jnp.float32),
            pltpu.VMEM((BW,), jnp.float32),
            pltpu.VMEM((BW,), jnp.float32),
            pltpu.VMEM((32,), jnp.float32),
            pltpu.VMEM((16, BW), jnp.float32),
            pltpu.VMEM_SHARED((N,), jnp.float32),
            pltpu.VMEM_SHARED((N,), jnp.float32),
            pltpu.VMEM_SHARED((N,), jnp.float32),
            pltpu.SemaphoreType.DMA,
            pltpu.SemaphoreType.DMA,
            pltpu.SemaphoreType.DMA,
            pltpu.SemaphoreType.DMA,
            pltpu.SemaphoreType.DMA,
            pltpu.SemaphoreType.DMA,
        ],
    )
    def gib_sc(px_hbm, py_hbm, pz_hbm, idx_hbm, qx_hbm, qy_hbm, qz_hbm,
               c_hbm, out_hbm,
               idxv, sxv, syv, szv, qxv, qyv, qzv, cv, outv,
               px_sh, py_sh, pz_sh, *sems):
        w = lax.axis_index("s") * 2 + lax.axis_index("c")
        base = w * BW

        @pl.when(lax.axis_index("s") == 0)
        def _stage():
            pltpu.sync_copy(px_hbm, px_sh)
            pltpu.sync_copy(py_hbm, py_sh)
            pltpu.sync_copy(pz_hbm, pz_sh)

        pltpu.sync_copy(idx_hbm.at[pl.ds(w * IDX_ROWS, IDX_ROWS)], idxv)
        pltpu.sync_copy(qx_hbm.at[pl.ds(base, BW)], qxv)
        pltpu.sync_copy(qy_hbm.at[pl.ds(base, BW)], qyv)
        pltpu.sync_copy(qz_hbm.at[pl.ds(base, BW)], qzv)
        pltpu.sync_copy(c_hbm, cv)
        plsc.subcore_barrier()

        shs = (px_sh, py_sh, pz_sh)
        dsts = (sxv, syv, szv)

        def fire(c):
            par = 3 * (c % 2)
            for d in range(3):
                for r in range(RPC):
                    j = c * RPC + r
                    pltpu.async_copy(shs[d].at[idxv.at[j]],
                                     dsts[d].at[pl.ds(j * 128, 128)],
                                     sems[par + d])

        def drain(c):
            par = 3 * (c % 2)
            for d in range(3):
                pltpu.make_async_copy(
                    px_hbm.at[pl.ds(0, CW)],
                    dsts[d].at[pl.ds(c * CW, CW)],
                    sems[par + d]).wait()

        zf = jnp.zeros((16,), jnp.float32)
        c_lo = cv[pl.ds(0, 16)]
        c_hi = cv[pl.ds(16, 16)]
        C = [c_lo[i] for i in range(16)] + [c_hi[i] for i in range(8)]

        def group(gi_abs, carry):
            qb = gi_abs * 16
            qxg = qxv[pl.ds(qb, 16)]
            qyg = qyv[pl.ds(qb, 16)]
            qzg = qzv[pl.ds(qb, 16)]
            accs = [zf for _ in range(12)]
            cbase = lax.mul(lax.div(qb, QC), CW)
            qoff = lax.rem(qb, QC)
            for k in range(K):
                sl = pl.ds(cbase + k * QC + qoff, 16)
                sx = sxv[sl]
                sy = syv[sl]
                sz = szv[sl]
                dx = sx - qxg
                dy = sy - qyg
                dz = sz - qzg
                dxy2 = dx * dx + dy * dy
                dz2 = dz * dz
                adz = jnp.abs(dz) + 1e-6
                t = dxy2 + 1e-8
                ti = lax.bitcast_convert_type(t, jnp.int32)
                yi = jnp.int32(0x5F3759DF) - lax.shift_right_logical(ti, 1)
                y = lax.bitcast_convert_type(yi, jnp.float32)
                for _ in range(2):
                    y = y * (1.5 - 0.5 * t * y * y)
                ratio = (t * y) / adz
                for i in range(4):
                    d0 = dxy2 - C[i]
                    accs[i] = accs[i] + jnp.exp(-(d0 * d0) * C[4 + i])
                for i in range(4):
                    d0 = ratio - C[8 + i]
                    accs[4 + i] = accs[4 + i] + jnp.exp(-(d0 * d0) * C[12 + i])
                for i in range(4):
                    accs[8 + i] = accs[8 + i] + jnp.exp(
                        -(dxy2 * C[16 + i] + dz2 * C[20 + i]))
            for gi in range(12):
                outv[gi, pl.ds(qb, 16)] = accs[gi]
            for gi in range(12, 16):
                outv[gi, pl.ds(qb, 16)] = zf
            return carry

        fire(0)
        for c in range(NCH):
            drain(c)
            if c + 1 < NCH:
                fire(c + 1)
            lax.fori_loop(c * G, (c + 1) * G, group, 0)

        pltpu.sync_copy(outv, out_hbm.at[w])

    return gib_sc


_GIB_SC = _build_sc_kernel()


def _mm_body(q_ref, l_ref, o_ref):
    o_ref[...] = jnp.dot(q_ref[...], l_ref[...],
                         preferred_element_type=jnp.float32)


def kernel(points, q_coords, support_idxs, mc_points,
           cy_params, cone_params, disk_params, lambdas):
    del mc_points
    px = points[:, 0]
    py = points[:, 1]
    pz = points[:, 2]
    qpad = jnp.pad(q_coords, ((0, MP - M), (0, 0)))
    qx = qpad[:, 0]
    qy = qpad[:, 1]
    qz = qpad[:, 2]
    idx = jnp.pad(support_idxs.astype(jnp.int32), ((0, MP - M), (0, 0)))
    idx = idx.reshape(NW, NCH, QC, K).transpose(0, 1, 3, 2)
    idx = idx.reshape(NW * IDX_ROWS, 128)
    consts = jnp.concatenate([
        cy_params[:, 0] ** 2,
        1.0 / (2.0 * cy_params[:, 1] ** 2 + 1e-8),
        cone_params[:, 0],
        1.0 / (2.0 * cone_params[:, 1] ** 2 + 1e-8),
        1.0 / (2.0 * disk_params[:, 0] ** 2 + 1e-8),
        1.0 / (2.0 * disk_params[:, 1] ** 2 + 1e-8),
        jnp.zeros((8,), jnp.float32),
    ])
    q_out_t = _GIB_SC(px, py, pz, idx, qx, qy, qz, consts)
    q_out = q_out_t.transpose(0, 2, 1).reshape(MP, 16)
    lam_pad = jnp.pad(lambdas, ((0, 16 - lambdas.shape[0]), (0, 0)))
    out = pl.pallas_call(
        _mm_body,
        out_shape=jax.ShapeDtypeStruct((MP, OBS), jnp.float32),
    )(q_out, lam_pad)
    return out[:M]


# speedup vs baseline: 1.0236x; 1.0236x over previous
import functools

import jax
import jax.numpy as jnp
from jax import lax
from jax.experimental import pallas as pl
from jax.experimental.pallas import tpu as pltpu
from jax.experimental.pallas import tpu_sc as plsc

N = 100000
M = 20000
K = 16
OBS = 16
NW = 32
MP = 20480
BW = MP // NW
QC = 160
NCH = BW // QC
CW = QC * K
G = QC // 16
ROWS_W = BW * K
IDX_ROWS = ROWS_W // 128
RPC = CW // 128


def _build_sc_kernel():
    mesh = plsc.VectorSubcoreMesh(core_axis_name="c", subcore_axis_name="s")

    @functools.partial(
        pl.kernel,
        out_type=jax.ShapeDtypeStruct((NW, 16, BW), jnp.float32),
        mesh=mesh,
        scratch_types=[
            pltpu.VMEM((IDX_ROWS, 128), jnp.int32),
            pltpu.VMEM((ROWS_W,), jnp.float32),
            pltpu.VMEM((ROWS_W,), jnp.float32),
            pltpu.VMEM((ROWS_W,), jnp.float32),
            pltpu.VMEM((BW,), jnp.float32),
            pltpu.VMEM((BW,), jnp.float32),
            pltpu.VMEM((BW,), jnp.float32),
            pltpu.VMEM((32,), jnp.float32),
            pltpu.VMEM((16, BW), jnp.float32),
            pltpu.VMEM_SHARED((N,), jnp.float32),
            pltpu.VMEM_SHARED((N,), jnp.float32),
            pltpu.VMEM_SHARED((N,), jnp.float32),
            pltpu.SemaphoreType.DMA,
            pltpu.SemaphoreType.DMA,
            pltpu.SemaphoreType.DMA,
            pltpu.SemaphoreType.DMA,
            pltpu.SemaphoreType.DMA,
            pltpu.SemaphoreType.DMA,
        ],
    )
    def gib_sc(px_hbm, py_hbm, pz_hbm, idx_hbm, qx_hbm, qy_hbm, qz_hbm,
               c_hbm, out_hbm,
               idxv, sxv, syv, szv, qxv, qyv, qzv, cv, outv,
               px_sh, py_sh, pz_sh, *sems):
        w = lax.axis_index("s") * 2 + lax.axis_index("c")
        base = w * BW

        @pl.when(lax.axis_index("s") == 0)
        def _stage():
            pltpu.sync_copy(px_hbm, px_sh)
            pltpu.sync_copy(py_hbm, py_sh)
            pltpu.sync_copy(pz_hbm, pz_sh)

        pltpu.sync_copy(idx_hbm.at[pl.ds(w * IDX_ROWS, IDX_ROWS)], idxv)
        pltpu.sync_copy(qx_hbm.at[pl.ds(base, BW)], qxv)
        pltpu.sync_copy(qy_hbm.at[pl.ds(base, BW)], qyv)
        pltpu.sync_copy(qz_hbm.at[pl.ds(base, BW)], qzv)
        pltpu.sync_copy(c_hbm, cv)
        plsc.subcore_barrier()

        shs = (px_sh, py_sh, pz_sh)
        dsts = (sxv, syv, szv)

        def fire(c):
            par = 3 * (c % 2)
            for d in range(3):
                for r in range(RPC):
                    j = c * RPC + r
                    pltpu.async_copy(shs[d].at[idxv.at[j]],
                                     dsts[d].at[pl.ds(j * 128, 128)],
                                     sems[par + d])

        def drain(c):
            par = 3 * (c % 2)
            for d in range(3):
                pltpu.make_async_copy(
                    px_hbm.at[pl.ds(0, CW)],
                    dsts[d].at[pl.ds(c * CW, CW)],
                    sems[par + d]).wait()

        zf = jnp.zeros((16,), jnp.float32)
        c_lo = cv[pl.ds(0, 16)]
        c_hi = cv[pl.ds(16, 16)]
        C = [c_lo[i] for i in range(16)] + [c_hi[i] for i in range(8)]

        def group(gi_abs, carry):
            qb = gi_abs * 16
            qxg = qxv[pl.ds(qb, 16)]
            qyg = qyv[pl.ds(qb, 16)]
            qzg = qzv[pl.ds(qb, 16)]
            accs = [zf for _ in range(12)]
            cbase = lax.mul(lax.div(qb, QC), CW)
            qoff = lax.rem(qb, QC)
            for k in range(K):
                sl = pl.ds(cbase + k * QC + qoff, 16)
                sx = sxv[sl]
                sy = syv[sl]
                sz = szv[sl]
                dx = sx - qxg
                dy = sy - qyg
                dz = sz - qzg
                dxy2 = dx * dx + dy * dy
                dz2 = dz * dz
                adz = jnp.abs(dz) + 1e-6
                t = dxy2 + 1e-8
                ti = lax.bitcast_convert_type(t, jnp.int32)
                yi = jnp.int32(0x5F3759DF) - lax.shift_right_logical(ti, 1)
                y = lax.bitcast_convert_type(yi, jnp.float32)
                for _ in range(2):
                    y = y * (1.5 - 0.5 * t * y * y)
                ratio = (t * y) / adz
                for i in range(4):
                    d0 = dxy2 - C[i]
                    accs[i] = accs[i] + jnp.exp(-(d0 * d0) * C[4 + i])
                for i in range(4):
                    d0 = ratio - C[8 + i]
                    accs[4 + i] = accs[4 + i] + jnp.exp(-(d0 * d0) * C[12 + i])
                for i in range(4):
                    accs[8 + i] = accs[8 + i] + jnp.exp(
                        -(dxy2 * C[16 + i] + dz2 * C[20 + i]))
            for gi in range(12):
                outv[gi, pl.ds(qb, 16)] = accs[gi]
            for gi in range(12, 16):
                outv[gi, pl.ds(qb, 16)] = zf
            return carry

        fire(0)
        for c in range(NCH):
            drain(c)
            if c + 1 < NCH:
                fire(c + 1)
            lax.fori_loop(c * G, (c + 1) * G, group, 0)

        pltpu.sync_copy(outv, out_hbm.at[w])

    return gib_sc


_GIB_SC = _build_sc_kernel()


def _mm_body(q_ref, l_ref, o_ref):
    o_ref[...] = jnp.dot(q_ref[...], l_ref[...],
                         preferred_element_type=jnp.float32)


def kernel(points, q_coords, support_idxs, mc_points,
           cy_params, cone_params, disk_params, lambdas):
    del mc_points
    px = points[:, 0]
    py = points[:, 1]
    pz = points[:, 2]
    qpad = jnp.pad(q_coords, ((0, MP - M), (0, 0)))
    qx = qpad[:, 0]
    qy = qpad[:, 1]
    qz = qpad[:, 2]
    idx = jnp.pad(support_idxs.astype(jnp.int32), ((0, MP - M), (0, 0)))
    idx = idx.reshape(NW, NCH, QC, K).transpose(0, 1, 3, 2)
    idx = idx.reshape(NW * IDX_ROWS, 128)
    consts = jnp.concatenate([
        cy_params[:, 0] ** 2,
        1.0 / (2.0 * cy_params[:, 1] ** 2 + 1e-8),
        cone_params[:, 0],
        1.0 / (2.0 * cone_params[:, 1] ** 2 + 1e-8),
        1.0 / (2.0 * disk_params[:, 0] ** 2 + 1e-8),
        1.0 / (2.0 * disk_params[:, 1] ** 2 + 1e-8),
        jnp.zeros((8,), jnp.float32),
    ])
    q_out_t = _GIB_SC(px, py, pz, idx, qx, qy, qz, consts)
    q_out = q_out_t.transpose(0, 2, 1).reshape(MP, 16)
    lam_pad = jnp.pad(lambdas, ((0, 16 - lambdas.shape[0]), (0, 0)))
    out = pl.pallas_call(
        _mm_body,
        out_shape=jax.ShapeDtypeStruct((MP, OBS), jnp.float32),
    )(q_out, lam_pad)
    return out[:M]


# speedup vs baseline: 1.1414x; 1.1151x over previous
import functools

import jax
import jax.numpy as jnp
from jax import lax
from jax.experimental import pallas as pl
from jax.experimental.pallas import tpu as pltpu
from jax.experimental.pallas import tpu_sc as plsc

N = 100000
M = 20000
K = 16
OBS = 16
NW = 32
MP = 20480
BW = MP // NW
QC = 128
NCH = BW // QC
CW = QC * K
G = QC // 16
ROWS_W = BW * K
IDX_ROWS = ROWS_W // 128
RPC = CW // 128


def _build_sc_kernel():
    mesh = plsc.VectorSubcoreMesh(core_axis_name="c", subcore_axis_name="s")

    @functools.partial(
        pl.kernel,
        out_type=jax.ShapeDtypeStruct((NW, 16, BW), jnp.float32),
        mesh=mesh,
        scratch_types=[
            pltpu.VMEM((IDX_ROWS, 128), jnp.int32),
            pltpu.VMEM((ROWS_W,), jnp.float32),
            pltpu.VMEM((ROWS_W,), jnp.float32),
            pltpu.VMEM((ROWS_W,), jnp.float32),
            pltpu.VMEM((BW,), jnp.float32),
            pltpu.VMEM((BW,), jnp.float32),
            pltpu.VMEM((BW,), jnp.float32),
            pltpu.VMEM((32,), jnp.float32),
            pltpu.VMEM((12, 16), jnp.float32),
            pltpu.VMEM((16, BW), jnp.float32),
            pltpu.VMEM_SHARED((N,), jnp.float32),
            pltpu.VMEM_SHARED((N,), jnp.float32),
            pltpu.VMEM_SHARED((N,), jnp.float32),
            pltpu.SemaphoreType.DMA,
            pltpu.SemaphoreType.DMA,
            pltpu.SemaphoreType.DMA,
            pltpu.SemaphoreType.DMA,
            pltpu.SemaphoreType.DMA,
            pltpu.SemaphoreType.DMA,
        ],
    )
    def gib_sc(px_hbm, py_hbm, pz_hbm, idx_hbm, qx_hbm, qy_hbm, qz_hbm,
               c_hbm, lam_hbm, out_hbm,
               idxv, sxv, syv, szv, qxv, qyv, qzv, cv, lamv, outv,
               px_sh, py_sh, pz_sh, *sems):
        w = lax.axis_index("s") * 2 + lax.axis_index("c")
        base = w * BW

        @pl.when(lax.axis_index("s") == 0)
        def _stage():
            pltpu.sync_copy(px_hbm, px_sh)
            pltpu.sync_copy(py_hbm, py_sh)
            pltpu.sync_copy(pz_hbm, pz_sh)

        pltpu.sync_copy(idx_hbm.at[pl.ds(w * IDX_ROWS, IDX_ROWS)], idxv)
        pltpu.sync_copy(qx_hbm.at[pl.ds(base, BW)], qxv)
        pltpu.sync_copy(qy_hbm.at[pl.ds(base, BW)], qyv)
        pltpu.sync_copy(qz_hbm.at[pl.ds(base, BW)], qzv)
        pltpu.sync_copy(c_hbm, cv)
        pltpu.sync_copy(lam_hbm, lamv)
        plsc.subcore_barrier()

        shs = (px_sh, py_sh, pz_sh)
        dsts = (sxv, syv, szv)

        def fire(c):
            par = 3 * (c % 2)
            for d in range(3):
                for r in range(RPC):
                    j = c * RPC + r
                    pltpu.async_copy(shs[d].at[idxv.at[j]],
                                     dsts[d].at[pl.ds(j * 128, 128)],
                                     sems[par + d])

        def drain(c):
            par = 3 * (c % 2)
            for d in range(3):
                pltpu.make_async_copy(
                    px_hbm.at[pl.ds(0, CW)],
                    dsts[d].at[pl.ds(c * CW, CW)],
                    sems[par + d]).wait()

        zf = jnp.zeros((16,), jnp.float32)
        c_lo = cv[pl.ds(0, 16)]
        c_hi = cv[pl.ds(16, 16)]
        C = [c_lo[i] for i in range(16)] + [c_hi[i] for i in range(8)]
        lrows = [lamv[gi] for gi in range(12)]
        L = [[lrows[gi][o] for o in range(16)] for gi in range(12)]

        def group(gi_abs, carry):
            qb = gi_abs * 16
            qxg = qxv[pl.ds(qb, 16)]
            qyg = qyv[pl.ds(qb, 16)]
            qzg = qzv[pl.ds(qb, 16)]
            accs = [zf for _ in range(12)]
            cbase = lax.mul(lax.div(qb, QC), CW)
            qoff = lax.rem(qb, QC)
            for k in range(K):
                sl = pl.ds(cbase + k * QC + qoff, 16)
                sx = sxv[sl]
                sy = syv[sl]
                sz = szv[sl]
                dx = sx - qxg
                dy = sy - qyg
                dz = sz - qzg
                dxy2 = dx * dx + dy * dy
                dz2 = dz * dz
                adz = jnp.abs(dz) + 1e-6
                t = dxy2 + 1e-8
                ti = lax.bitcast_convert_type(t, jnp.int32)
                yi = jnp.int32(0x5F3759DF) - lax.shift_right_logical(ti, 1)
                y = lax.bitcast_convert_type(yi, jnp.float32)
                for _ in range(2):
                    y = y * (1.5 - 0.5 * t * y * y)
                ratio = (t * y) / adz
                for i in range(4):
                    d0 = dxy2 - C[i]
                    accs[i] = accs[i] + jnp.exp(-(d0 * d0) * C[4 + i])
                for i in range(4):
                    d0 = ratio - C[8 + i]
                    accs[4 + i] = accs[4 + i] + jnp.exp(-(d0 * d0) * C[12 + i])
                for i in range(4):
                    accs[8 + i] = accs[8 + i] + jnp.exp(
                        -(dxy2 * C[16 + i] + dz2 * C[20 + i]))
            for o in range(16):
                r = accs[0] * L[0][o]
                for gi in range(1, 12):
                    r = r + accs[gi] * L[gi][o]
                outv[o, pl.ds(qb, 16)] = r
            return carry

        fire(0)
        for c in range(NCH):
            drain(c)
            if c + 1 < NCH:
                fire(c + 1)
            lax.fori_loop(c * G, (c + 1) * G, group, 0)

        pltpu.sync_copy(outv, out_hbm.at[w])

    return gib_sc


_GIB_SC = _build_sc_kernel()


def _tr_body(q_ref, o_ref):
    o_ref[...] = jnp.swapaxes(q_ref[...], 1, 2)


def kernel(points, q_coords, support_idxs, mc_points,
           cy_params, cone_params, disk_params, lambdas):
    del mc_points
    px = points[:, 0]
    py = points[:, 1]
    pz = points[:, 2]
    qpad = jnp.pad(q_coords, ((0, MP - M), (0, 0)))
    qx = qpad[:, 0]
    qy = qpad[:, 1]
    qz = qpad[:, 2]
    idx = jnp.pad(support_idxs.astype(jnp.int32), ((0, MP - M), (0, 0)))
    idx = idx.reshape(NW, NCH, QC, K).transpose(0, 1, 3, 2)
    idx = idx.reshape(NW * IDX_ROWS, 128)
    consts = jnp.concatenate([
        cy_params[:, 0] ** 2,
        1.0 / (2.0 * cy_params[:, 1] ** 2 + 1e-8),
        cone_params[:, 0],
        1.0 / (2.0 * cone_params[:, 1] ** 2 + 1e-8),
        1.0 / (2.0 * disk_params[:, 0] ** 2 + 1e-8),
        1.0 / (2.0 * disk_params[:, 1] ** 2 + 1e-8),
        jnp.zeros((8,), jnp.float32),
    ])
    q_out_t = _GIB_SC(px, py, pz, idx, qx, qy, qz, consts, lambdas)
    out = pl.pallas_call(
        _tr_body,
        out_shape=jax.ShapeDtypeStruct((NW, BW, OBS), jnp.float32),
        grid=(2,),
        in_specs=[pl.BlockSpec((NW // 2, 16, BW), lambda i: (i, 0, 0))],
        out_specs=pl.BlockSpec((NW // 2, BW, OBS), lambda i: (i, 0, 0)),
    )(q_out_t)
    return out.reshape(MP, OBS)[:M]


# speedup vs baseline: 1.1463x; 1.0043x over previous
import functools

import jax
import jax.numpy as jnp
from jax import lax
from jax.experimental import pallas as pl
from jax.experimental.pallas import tpu as pltpu
from jax.experimental.pallas import tpu_sc as plsc

N = 100000
M = 20000
K = 16
OBS = 16
NW = 32
MP = 20480
BW = MP // NW
QC = 128
NCH = BW // QC
CW = QC * K
G = QC // 16
ROWS_W = BW * K
IDX_ROWS = ROWS_W // 128
RPC = CW // 128


def _build_sc_kernel():
    mesh = plsc.VectorSubcoreMesh(core_axis_name="c", subcore_axis_name="s")

    @functools.partial(
        pl.kernel,
        out_type=jax.ShapeDtypeStruct((NW, 16, BW), jnp.float32),
        mesh=mesh,
        scratch_types=[
            pltpu.VMEM((IDX_ROWS, 128), jnp.int32),
            pltpu.VMEM((ROWS_W,), jnp.float32),
            pltpu.VMEM((ROWS_W,), jnp.float32),
            pltpu.VMEM((ROWS_W,), jnp.float32),
            pltpu.VMEM((BW,), jnp.float32),
            pltpu.VMEM((BW,), jnp.float32),
            pltpu.VMEM((BW,), jnp.float32),
            pltpu.VMEM((32,), jnp.float32),
            pltpu.VMEM((12, 16), jnp.float32),
            pltpu.VMEM((16, BW), jnp.float32),
            pltpu.VMEM_SHARED((N,), jnp.float32),
            pltpu.VMEM_SHARED((N,), jnp.float32),
            pltpu.VMEM_SHARED((N,), jnp.float32),
            pltpu.SemaphoreType.DMA,
            pltpu.SemaphoreType.DMA,
            pltpu.SemaphoreType.DMA,
            pltpu.SemaphoreType.DMA,
            pltpu.SemaphoreType.DMA,
            pltpu.SemaphoreType.DMA,
            pltpu.SemaphoreType.DMA,
        ],
    )
    def gib_sc(px_hbm, py_hbm, pz_hbm, idx_hbm, qx_hbm, qy_hbm, qz_hbm,
               c_hbm, lam_hbm, out_hbm,
               idxv, sxv, syv, szv, qxv, qyv, qzv, cv, lamv, outv,
               px_sh, py_sh, pz_sh, *sems):
        w = lax.axis_index("s") * 2 + lax.axis_index("c")
        base = w * BW

        @pl.when(lax.axis_index("s") == 0)
        def _stage():
            pltpu.sync_copy(px_hbm, px_sh)
            pltpu.sync_copy(py_hbm, py_sh)
            pltpu.sync_copy(pz_hbm, pz_sh)

        pltpu.sync_copy(idx_hbm.at[pl.ds(w * IDX_ROWS, IDX_ROWS)], idxv)
        pltpu.sync_copy(qx_hbm.at[pl.ds(base, BW)], qxv)
        pltpu.sync_copy(qy_hbm.at[pl.ds(base, BW)], qyv)
        pltpu.sync_copy(qz_hbm.at[pl.ds(base, BW)], qzv)
        pltpu.sync_copy(c_hbm, cv)
        pltpu.sync_copy(lam_hbm, lamv)
        plsc.subcore_barrier()

        shs = (px_sh, py_sh, pz_sh)
        dsts = (sxv, syv, szv)

        def fire(c):
            par = 3 * (c % 2)
            for d in range(3):
                for r in range(RPC):
                    j = c * RPC + r
                    pltpu.async_copy(shs[d].at[idxv.at[j]],
                                     dsts[d].at[pl.ds(j * 128, 128)],
                                     sems[par + d])

        def drain(c):
            par = 3 * (c % 2)
            for d in range(3):
                pltpu.make_async_copy(
                    px_hbm.at[pl.ds(0, CW)],
                    dsts[d].at[pl.ds(c * CW, CW)],
                    sems[par + d]).wait()

        zf = jnp.zeros((16,), jnp.float32)
        c_lo = cv[pl.ds(0, 16)]
        c_hi = cv[pl.ds(16, 16)]
        C = [c_lo[i] for i in range(16)] + [c_hi[i] for i in range(8)]
        lrows = [lamv[gi] for gi in range(12)]
        L = [[lrows[gi][o] for o in range(16)] for gi in range(12)]

        def group(gi_abs, carry):
            qb = gi_abs * 16
            qxg = qxv[pl.ds(qb, 16)]
            qyg = qyv[pl.ds(qb, 16)]
            qzg = qzv[pl.ds(qb, 16)]
            accs = [zf for _ in range(12)]
            cbase = lax.mul(lax.div(qb, QC), CW)
            qoff = lax.rem(qb, QC)
            for k in range(K):
                sl = pl.ds(cbase + k * QC + qoff, 16)
                sx = sxv[sl]
                sy = syv[sl]
                sz = szv[sl]
                dx = sx - qxg
                dy = sy - qyg
                dz = sz - qzg
                dxy2 = dx * dx + dy * dy
                dz2 = dz * dz
                adz = jnp.abs(dz) + 1e-6
                t = dxy2 + 1e-8
                ti = lax.bitcast_convert_type(t, jnp.int32)
                yi = jnp.int32(0x5F3759DF) - lax.shift_right_logical(ti, 1)
                y = lax.bitcast_convert_type(yi, jnp.float32)
                for _ in range(2):
                    y = y * (1.5 - 0.5 * t * y * y)
                ratio = (t * y) / adz
                for i in range(4):
                    d0 = dxy2 - C[i]
                    accs[i] = accs[i] + jnp.exp(-(d0 * d0) * C[4 + i])
                for i in range(4):
                    d0 = ratio - C[8 + i]
                    accs[4 + i] = accs[4 + i] + jnp.exp(-(d0 * d0) * C[12 + i])
                for i in range(4):
                    accs[8 + i] = accs[8 + i] + jnp.exp(
                        -(dxy2 * C[16 + i] + dz2 * C[20 + i]))
            for o in range(16):
                r = accs[0] * L[0][o]
                for gi in range(1, 12):
                    r = r + accs[gi] * L[gi][o]
                outv[o, pl.ds(qb, 16)] = r
            return carry

        fire(0)
        for c in range(NCH):
            drain(c)
            if c + 1 < NCH:
                fire(c + 1)
            lax.fori_loop(c * G, (c + 1) * G, group, 0)
            qsl = pl.ds(c * QC, QC)
            pltpu.async_copy(outv.at[pl.ds(0, 16), qsl],
                             out_hbm.at[w, pl.ds(0, 16), qsl],
                             sems[6])
        for c in range(NCH):
            pltpu.make_async_copy(
                outv.at[pl.ds(0, 16), pl.ds(c * QC, QC)],
                out_hbm.at[w, pl.ds(0, 16), pl.ds(c * QC, QC)],
                sems[6]).wait()

    return gib_sc


_GIB_SC = _build_sc_kernel()


def _tr_body(q_ref, o_ref):
    o_ref[...] = jnp.swapaxes(q_ref[...], 1, 2)


def kernel(points, q_coords, support_idxs, mc_points,
           cy_params, cone_params, disk_params, lambdas):
    del mc_points
    px = points[:, 0]
    py = points[:, 1]
    pz = points[:, 2]
    qpad = jnp.pad(q_coords, ((0, MP - M), (0, 0)))
    qx = qpad[:, 0]
    qy = qpad[:, 1]
    qz = qpad[:, 2]
    idx = jnp.pad(support_idxs.astype(jnp.int32), ((0, MP - M), (0, 0)))
    idx = idx.reshape(NW, NCH, QC, K).transpose(0, 1, 3, 2)
    idx = idx.reshape(NW * IDX_ROWS, 128)
    consts = jnp.concatenate([
        cy_params[:, 0] ** 2,
        1.0 / (2.0 * cy_params[:, 1] ** 2 + 1e-8),
        cone_params[:, 0],
        1.0 / (2.0 * cone_params[:, 1] ** 2 + 1e-8),
        1.0 / (2.0 * disk_params[:, 0] ** 2 + 1e-8),
        1.0 / (2.0 * disk_params[:, 1] ** 2 + 1e-8),
        jnp.zeros((8,), jnp.float32),
    ])
    q_out_t = _GIB_SC(px, py, pz, idx, qx, qy, qz, consts, lambdas)
    out = pl.pallas_call(
        _tr_body,
        out_shape=jax.ShapeDtypeStruct((NW, BW, OBS), jnp.float32),
        grid=(2,),
        in_specs=[pl.BlockSpec((NW // 2, 16, BW), lambda i: (i, 0, 0))],
        out_specs=pl.BlockSpec((NW // 2, BW, OBS), lambda i: (i, 0, 0)),
    )(q_out_t)
    return out.reshape(MP, OBS)[:M]
